# baseline (speedup 1.0000x reference)
import functools

import jax
import jax.numpy as jnp
from jax import lax
from jax.experimental import pallas as pl
from jax.experimental.pallas import tpu as pltpu
from jax.experimental.pallas import tpu_sc as plsc

M, N, DV = 4096, 8192, 6
MAX_ITER = 8
ALPHA = 0.8
CLAMP = 20.0
MAXDC = 12
L = 16
NW = 32
GROUPS = M // (2 * L)
BIG = 3.0e38


def _two_min_tree(avs):
    pairs = []
    for i in range(0, len(avs) - 1, 2):
        a, b = avs[i], avs[i + 1]
        pairs.append((jnp.minimum(a, b), jnp.maximum(a, b)))
    if len(avs) % 2:
        big = jnp.full((L,), BIG, jnp.float32)
        pairs.append((avs[-1], big))
    while len(pairs) > 1:
        nxt = []
        for i in range(0, len(pairs) - 1, 2):
            (m1a, m2a), (m1b, m2b) = pairs[i], pairs[i + 1]
            nxt.append((jnp.minimum(m1a, m1b),
                        jnp.minimum(jnp.maximum(m1a, m1b),
                                    jnp.minimum(m2a, m2b))))
        if len(pairs) % 2:
            nxt.append(pairs[-1])
        pairs = nxt
    return pairs[0]


def _tree_mul(xs):
    while len(xs) > 1:
        nxt = [xs[i] * xs[i + 1] for i in range(0, len(xs) - 1, 2)]
        if len(xs) % 2:
            nxt.append(xs[-1])
        xs = nxt
    return xs[0]


def _sc_decode(channel, s_sign, dc, vslp, B):
    mesh = plsc.VectorSubcoreMesh(core_axis_name="c", subcore_axis_name="s")
    rounds = B // NW
    half = MAXDC // 2

    @functools.partial(
        pl.kernel,
        mesh=mesh,
        compiler_params=pltpu.CompilerParams(needs_layout_passes=False),
        out_type=[
            jax.ShapeDtypeStruct((B * MAX_ITER, N), jnp.float32),
            jax.ShapeDtypeStruct((B, MAX_ITER * L), jnp.float32),
        ],
        scratch_types=[
            pltpu.VMEM((MAXDC * M,), jnp.float32),
            pltpu.VMEM((N + L,), jnp.float32),
            pltpu.VMEM((N + L,), jnp.float32),
            pltpu.VMEM((N,), jnp.float32),
            pltpu.VMEM((M,), jnp.float32),
            pltpu.VMEM((M,), jnp.int32),
            pltpu.VMEM((half * M,), jnp.int32),
            pltpu.VMEM((MAX_ITER * L,), jnp.float32),
            pltpu.SemaphoreType.DMA,
        ],
    )
    def k(chan_hbm, ss_hbm, dc_hbm, vslp_hbm, tsnap_hbm, conv_hbm,
          C, t_a, t_b, chan_v, ss_v, dc_v, vsl_v, conv_s, snap_sem):
        wid = lax.axis_index("s") * 2 + lax.axis_index("c")
        pltpu.sync_copy(dc_hbm, dc_v)
        pltpu.sync_copy(vslp_hbm, vsl_v)

        c16 = jnp.full((L,), CLAMP, jnp.float32)
        one = jnp.full((L,), 1.0, jnp.float32)
        zero = jnp.full((L,), 0.0, jnp.float32)
        big = jnp.full((L,), BIG, jnp.float32)
        mask16 = jnp.full((L,), 0xFFFF, jnp.int32)
        sh16 = jnp.full((L,), 16, jnp.int32)
        eps = jnp.full((L,), 1e-9, jnp.float32)
        alpha = jnp.full((L,), ALPHA, jnp.float32)

        def init_t(dst):
            @plsc.parallel_loop(0, N // L, unroll=4)
            def _(i):
                dst[pl.ds(i * L, L)] = chan_v[pl.ds(i * L, L)]
            dst[pl.ds(N, L)] = one

        def chunk_pass(t_cur, t_next, parity, update):
            @plsc.parallel_loop(0, M // L, unroll=2, carry=one)
            def conv_out(g, conv_acc):
                cbase = g * L
                dcv = dc_v[pl.ds(cbase, L)]
                ssv = ss_v[pl.ds(cbase, L)]
                m11 = jnp.full((L,), MAXDC - 1, jnp.int32) < dcv
                ws, sas, avs, pps = [], [], [], []
                for jj in range(half):
                    w = vsl_v[pl.ds(jj * M + cbase, L)]
                    ws.append(w)
                    i0 = w & mask16
                    i1 = lax.shift_right_logical(w, sh16)
                    g0 = plsc.load_gather(t_cur, [i0])
                    g1 = plsc.load_gather(t_cur, [i1])
                    C0 = C[pl.ds((2 * jj) * M + cbase, L)]
                    C1 = C[pl.ds((2 * jj + 1) * M + cbase, L)]
                    v0 = jnp.minimum(jnp.maximum(g0 - C0, -c16), c16)
                    v1 = jnp.minimum(jnp.maximum(g1 - C1, -c16), c16)
                    s0 = jnp.where(v0 < 0.0, -one, one)
                    s1 = jnp.where(v1 < 0.0, -one, one)
                    a0 = jnp.abs(v0)
                    a1 = jnp.abs(v1)
                    if jj == half - 1:
                        a1 = jnp.where(m11, a1, big)
                    sas += [s0, s1]
                    avs += [a0, a1]
                    if parity:
                        p0 = jnp.where(g0 < 0.0, -one, one)
                        p1 = jnp.where(g1 < 0.0, -one, one)
                        pps.append(p0 * p1)
                mn1, mn2 = _two_min_tree(avs)
                if parity:
                    par = _tree_mul(pps)
                    ok = jnp.where(par == ssv, one, zero)
                    conv_acc = jnp.minimum(conv_acc, ok)
                if update:
                    S = ssv * _tree_mul(sas) * alpha
                    for jj in range(half):
                        w = ws[jj]
                        i0 = w & mask16
                        i1 = lax.shift_right_logical(w, sh16)
                        a0, a1 = avs[2 * jj], avs[2 * jj + 1]
                        ex0 = jnp.where(jnp.abs(a0 - mn1) < eps, mn2, mn1)
                        ex1 = jnp.where(jnp.abs(a1 - mn1) < eps, mn2, mn1)
                        cn0 = S * sas[2 * jj] * ex0
                        cn1 = S * sas[2 * jj + 1] * ex1
                        if jj == half - 1:
                            cn1 = jnp.where(m11, cn1, zero)
                        C[pl.ds((2 * jj) * M + cbase, L)] = cn0
                        C[pl.ds((2 * jj + 1) * M + cbase, L)] = cn1
                        plsc.addupdate_scatter(t_next, [i0], cn0)
                        plsc.addupdate_scatter(t_next, [i1], cn1)
                return conv_acc
            return conv_out

        def do_round(r, _):
            b = r * NW + wid
            pltpu.sync_copy(chan_hbm.at[b], chan_v)
            pltpu.sync_copy(ss_hbm.at[b], ss_v)

            @plsc.parallel_loop(0, MAXDC * M // L, unroll=4)
            def _(i):
                C[pl.ds(i * L, L)] = zero

            init_t(t_a)
            init_t(t_b)
            chunk_pass(t_a, t_b, parity=False, update=True)
            cur, nxt = t_b, t_a
            prev_desc = None
            for kk in range(1, MAX_ITER):
                desc = pltpu.async_copy(
                    cur.at[pl.ds(0, N)], tsnap_hbm.at[b * MAX_ITER + kk - 1],
                    snap_sem)
                if prev_desc is not None:
                    prev_desc.wait()
                init_t(nxt)
                conv = chunk_pass(cur, nxt, parity=True, update=True)
                conv_s[pl.ds((kk - 1) * L, L)] = conv
                prev_desc = desc
                cur, nxt = nxt, cur
            desc = pltpu.async_copy(
                cur.at[pl.ds(0, N)], tsnap_hbm.at[b * MAX_ITER + MAX_ITER - 1],
                snap_sem)
            conv = chunk_pass(cur, nxt, parity=True, update=False)
            conv_s[pl.ds((MAX_ITER - 1) * L, L)] = conv
            pltpu.sync_copy(conv_s, conv_hbm.at[b])
            prev_desc.wait()
            desc.wait()
            return 0

        lax.fori_loop(0, rounds, do_round, 0)

    return k(channel, s_sign, dc, vslp)


_EPI_BB = 8


def _epilogue_body(tsnap_ref, conv_ref, marg_ref, hard_ref, convd_ref):
    conv = conv_ref[...]
    allc = jnp.min(conv, axis=0)
    it7 = jnp.arange(MAX_ITER, dtype=jnp.int32)
    ks = jnp.min(jnp.where((allc > 0.5) & (it7 < MAX_ITER - 1), it7, MAX_ITER - 1))
    snaps = tsnap_ref[...]
    sel = (it7 == ks).astype(jnp.float32)
    t = jnp.sum(snaps * sel[None, :, None], axis=1)
    marg_ref[...] = jax.nn.sigmoid(-t)
    hard_ref[...] = (t < 0).astype(jnp.int32)
    b = pl.program_id(0)
    conv8 = conv_ref[pl.ds(b * _EPI_BB, _EPI_BB), :]
    convd_ref[...] = jnp.sum(conv8 * sel[None, :], axis=1, keepdims=True)


def _epilogue(tsnap, conv):
    B = conv.shape[0]
    marg, hard, convd = pl.pallas_call(
        _epilogue_body,
        grid=(B // _EPI_BB,),
        in_specs=[
            pl.BlockSpec((_EPI_BB, MAX_ITER, N), lambda b: (b, 0, 0)),
            pl.BlockSpec((B, MAX_ITER), lambda b: (0, 0)),
        ],
        out_specs=[
            pl.BlockSpec((_EPI_BB, N), lambda b: (b, 0)),
            pl.BlockSpec((_EPI_BB, N), lambda b: (b, 0)),
            pl.BlockSpec((_EPI_BB, 1), lambda b: (b, 0)),
        ],
        out_shape=[
            jax.ShapeDtypeStruct((B, N), jnp.float32),
            jax.ShapeDtypeStruct((B, N), jnp.int32),
            jax.ShapeDtypeStruct((B, 1), jnp.float32),
        ],
    )(tsnap, conv)
    return marg, hard, convd.reshape(B) > 0.5


def kernel(syndrome, channel_llr, var_idx, var_adj, var_adj_mask,
           check_adj, check_adj_mask, pcm_dense):
    B = syndrome.shape[0]
    E = var_idx.shape[0]
    maxdc = check_adj.shape[1]
    s_sign = 1.0 - 2.0 * syndrome
    cmask = check_adj_mask
    dc = cmask.sum(axis=1).astype(jnp.int32)
    rowstart = jnp.cumsum(dc) - dc
    eid = rowstart[None, :] + jnp.arange(maxdc, dtype=jnp.int32)[:, None]
    valid = jnp.arange(maxdc, dtype=jnp.int32)[:, None] < dc[None, :]
    vslot = jnp.where(valid,
                      var_idx.astype(jnp.int32)[jnp.clip(eid, 0, E - 1)],
                      N)
    vs2 = vslot.reshape(maxdc // 2, 2, M)
    vslp = (vs2[:, 0, :] | (vs2[:, 1, :] << 16)).reshape(-1)

    tsnap_flat, conv_lanes = _sc_decode(channel_llr, s_sign, dc, vslp, B)
    tsnap = tsnap_flat.reshape(B, MAX_ITER, N)
    conv = conv_lanes.reshape(B, MAX_ITER, L).min(axis=2)
    return _epilogue(tsnap, conv)


# speedup vs baseline: 1.0784x; 1.0784x over previous
import functools

import jax
import jax.numpy as jnp
from jax import lax
from jax.experimental import pallas as pl
from jax.experimental.pallas import tpu as pltpu
from jax.experimental.pallas import tpu_sc as plsc

M, N, DV = 4096, 8192, 6
MAX_ITER = 8
ALPHA = 0.8
CLAMP = 20.0
MAXDC = 12
L = 16
NW = 32
GROUPS = M // (2 * L)
BIG = 3.0e38


def _two_min_tree(avs):
    pairs = []
    for i in range(0, len(avs) - 1, 2):
        a, b = avs[i], avs[i + 1]
        pairs.append((jnp.minimum(a, b), jnp.maximum(a, b)))
    if len(avs) % 2:
        big = jnp.full((L,), BIG, jnp.float32)
        pairs.append((avs[-1], big))
    while len(pairs) > 1:
        nxt = []
        for i in range(0, len(pairs) - 1, 2):
            (m1a, m2a), (m1b, m2b) = pairs[i], pairs[i + 1]
            nxt.append((jnp.minimum(m1a, m1b),
                        jnp.minimum(jnp.maximum(m1a, m1b),
                                    jnp.minimum(m2a, m2b))))
        if len(pairs) % 2:
            nxt.append(pairs[-1])
        pairs = nxt
    return pairs[0]


def _tree_mul(xs):
    while len(xs) > 1:
        nxt = [xs[i] * xs[i + 1] for i in range(0, len(xs) - 1, 2)]
        if len(xs) % 2:
            nxt.append(xs[-1])
        xs = nxt
    return xs[0]


def _sc_decode(channel, s_sign, dc, vslp, B):
    mesh = plsc.VectorSubcoreMesh(core_axis_name="c", subcore_axis_name="s")
    rounds = B // NW
    half = MAXDC // 2

    @functools.partial(
        pl.kernel,
        mesh=mesh,
        compiler_params=pltpu.CompilerParams(needs_layout_passes=False),
        out_type=[
            jax.ShapeDtypeStruct((B * MAX_ITER, N), jnp.float32),
            jax.ShapeDtypeStruct((B, MAX_ITER * L), jnp.float32),
        ],
        scratch_types=[
            pltpu.VMEM((MAXDC * M,), jnp.float32),
            pltpu.VMEM((N + L,), jnp.float32),
            pltpu.VMEM((N + L,), jnp.float32),
            pltpu.VMEM((N,), jnp.float32),
            pltpu.VMEM((M,), jnp.float32),
            pltpu.VMEM((M,), jnp.int32),
            pltpu.VMEM((half * M,), jnp.int32),
            pltpu.VMEM((MAX_ITER * L,), jnp.float32),
            pltpu.SemaphoreType.DMA,
        ],
    )
    def k(chan_hbm, ss_hbm, dc_hbm, vslp_hbm, tsnap_hbm, conv_hbm,
          C, t_a, t_b, chan_v, ss_v, dc_v, vsl_v, conv_s, snap_sem):
        wid = lax.axis_index("s") * 2 + lax.axis_index("c")
        pltpu.sync_copy(dc_hbm, dc_v)
        pltpu.sync_copy(vslp_hbm, vsl_v)

        c16 = jnp.full((L,), CLAMP, jnp.float32)
        one = jnp.full((L,), 1.0, jnp.float32)
        zero = jnp.full((L,), 0.0, jnp.float32)
        big = jnp.full((L,), BIG, jnp.float32)
        mask16 = jnp.full((L,), 0xFFFF, jnp.int32)
        sh16 = jnp.full((L,), 16, jnp.int32)
        eps = jnp.full((L,), 1e-9, jnp.float32)
        alpha = jnp.full((L,), ALPHA, jnp.float32)

        def init_t(dst):
            @plsc.parallel_loop(0, N // L, unroll=4)
            def _(i):
                dst[pl.ds(i * L, L)] = chan_v[pl.ds(i * L, L)]
            dst[pl.ds(N, L)] = one

        def chunk_pass(t_cur, t_next, parity, update):
            @plsc.parallel_loop(0, M // L, unroll=1, carry=one)
            def conv_out(g, conv_acc):
                cbase = g * L
                dcv = dc_v[pl.ds(cbase, L)]
                ssv = ss_v[pl.ds(cbase, L)]
                m11 = jnp.full((L,), MAXDC - 1, jnp.int32) < dcv
                ws, sas, avs, pps = [], [], [], []
                for jj in range(half):
                    w = vsl_v[pl.ds(jj * M + cbase, L)]
                    ws.append(w)
                    i0 = w & mask16
                    i1 = lax.shift_right_logical(w, sh16)
                    g0 = plsc.load_gather(t_cur, [i0])
                    g1 = plsc.load_gather(t_cur, [i1])
                    C0 = C[pl.ds((2 * jj) * M + cbase, L)]
                    C1 = C[pl.ds((2 * jj + 1) * M + cbase, L)]
                    v0 = jnp.minimum(jnp.maximum(g0 - C0, -c16), c16)
                    v1 = jnp.minimum(jnp.maximum(g1 - C1, -c16), c16)
                    s0 = jnp.where(v0 < 0.0, -one, one)
                    s1 = jnp.where(v1 < 0.0, -one, one)
                    a0 = jnp.abs(v0)
                    a1 = jnp.abs(v1)
                    if jj == half - 1:
                        a1 = jnp.where(m11, a1, big)
                    sas += [s0, s1]
                    avs += [a0, a1]
                    if parity:
                        p0 = jnp.where(g0 < 0.0, -one, one)
                        p1 = jnp.where(g1 < 0.0, -one, one)
                        pps.append(p0 * p1)
                mn1, mn2 = _two_min_tree(avs)
                if parity:
                    par = _tree_mul(pps)
                    ok = jnp.where(par == ssv, one, zero)
                    conv_acc = jnp.minimum(conv_acc, ok)
                if update:
                    S = ssv * _tree_mul(sas) * alpha
                    for jj in range(half):
                        w = ws[jj]
                        i0 = w & mask16
                        i1 = lax.shift_right_logical(w, sh16)
                        a0, a1 = avs[2 * jj], avs[2 * jj + 1]
                        ex0 = jnp.where(jnp.abs(a0 - mn1) < eps, mn2, mn1)
                        ex1 = jnp.where(jnp.abs(a1 - mn1) < eps, mn2, mn1)
                        cn0 = S * sas[2 * jj] * ex0
                        cn1 = S * sas[2 * jj + 1] * ex1
                        if jj == half - 1:
                            cn1 = jnp.where(m11, cn1, zero)
                        C[pl.ds((2 * jj) * M + cbase, L)] = cn0
                        C[pl.ds((2 * jj + 1) * M + cbase, L)] = cn1
                        plsc.addupdate_scatter(t_next, [i0], cn0)
                        plsc.addupdate_scatter(t_next, [i1], cn1)
                return conv_acc
            return conv_out

        def do_round(r, _):
            b = r * NW + wid
            pltpu.sync_copy(chan_hbm.at[b], chan_v)
            pltpu.sync_copy(ss_hbm.at[b], ss_v)

            @plsc.parallel_loop(0, MAXDC * M // L, unroll=4)
            def _(i):
                C[pl.ds(i * L, L)] = zero

            init_t(t_a)
            init_t(t_b)
            chunk_pass(t_a, t_b, parity=False, update=True)
            cur, nxt = t_b, t_a
            prev_desc = None
            for kk in range(1, MAX_ITER):
                desc = pltpu.async_copy(
                    cur.at[pl.ds(0, N)], tsnap_hbm.at[b * MAX_ITER + kk - 1],
                    snap_sem)
                if prev_desc is not None:
                    prev_desc.wait()
                init_t(nxt)
                conv = chunk_pass(cur, nxt, parity=True, update=True)
                conv_s[pl.ds((kk - 1) * L, L)] = conv
                prev_desc = desc
                cur, nxt = nxt, cur
            desc = pltpu.async_copy(
                cur.at[pl.ds(0, N)], tsnap_hbm.at[b * MAX_ITER + MAX_ITER - 1],
                snap_sem)
            conv = chunk_pass(cur, nxt, parity=True, update=False)
            conv_s[pl.ds((MAX_ITER - 1) * L, L)] = conv
            pltpu.sync_copy(conv_s, conv_hbm.at[b])
            prev_desc.wait()
            desc.wait()
            return 0

        lax.fori_loop(0, rounds, do_round, 0)

    return k(channel, s_sign, dc, vslp)


_EPI_BB = 8


def _epilogue_body(tsnap_ref, conv_ref, marg_ref, hard_ref, convd_ref):
    conv = conv_ref[...]
    allc = jnp.min(conv, axis=0)
    it7 = jnp.arange(MAX_ITER, dtype=jnp.int32)
    ks = jnp.min(jnp.where((allc > 0.5) & (it7 < MAX_ITER - 1), it7, MAX_ITER - 1))
    snaps = tsnap_ref[...]
    sel = (it7 == ks).astype(jnp.float32)
    t = jnp.sum(snaps * sel[None, :, None], axis=1)
    marg_ref[...] = jax.nn.sigmoid(-t)
    hard_ref[...] = (t < 0).astype(jnp.int32)
    b = pl.program_id(0)
    conv8 = conv_ref[pl.ds(b * _EPI_BB, _EPI_BB), :]
    convd_ref[...] = jnp.sum(conv8 * sel[None, :], axis=1, keepdims=True)


def _epilogue(tsnap, conv):
    B = conv.shape[0]
    marg, hard, convd = pl.pallas_call(
        _epilogue_body,
        grid=(B // _EPI_BB,),
        in_specs=[
            pl.BlockSpec((_EPI_BB, MAX_ITER, N), lambda b: (b, 0, 0)),
            pl.BlockSpec((B, MAX_ITER), lambda b: (0, 0)),
        ],
        out_specs=[
            pl.BlockSpec((_EPI_BB, N), lambda b: (b, 0)),
            pl.BlockSpec((_EPI_BB, N), lambda b: (b, 0)),
            pl.BlockSpec((_EPI_BB, 1), lambda b: (b, 0)),
        ],
        out_shape=[
            jax.ShapeDtypeStruct((B, N), jnp.float32),
            jax.ShapeDtypeStruct((B, N), jnp.int32),
            jax.ShapeDtypeStruct((B, 1), jnp.float32),
        ],
    )(tsnap, conv)
    return marg, hard, convd.reshape(B) > 0.5


def kernel(syndrome, channel_llr, var_idx, var_adj, var_adj_mask,
           check_adj, check_adj_mask, pcm_dense):
    B = syndrome.shape[0]
    E = var_idx.shape[0]
    maxdc = check_adj.shape[1]
    s_sign = 1.0 - 2.0 * syndrome
    cmask = check_adj_mask
    dc = cmask.sum(axis=1).astype(jnp.int32)
    rowstart = jnp.cumsum(dc) - dc
    eid = rowstart[None, :] + jnp.arange(maxdc, dtype=jnp.int32)[:, None]
    valid = jnp.arange(maxdc, dtype=jnp.int32)[:, None] < dc[None, :]
    vslot = jnp.where(valid,
                      var_idx.astype(jnp.int32)[jnp.clip(eid, 0, E - 1)],
                      N)
    vs2 = vslot.reshape(maxdc // 2, 2, M)
    vslp = (vs2[:, 0, :] | (vs2[:, 1, :] << 16)).reshape(-1)

    tsnap_flat, conv_lanes = _sc_decode(channel_llr, s_sign, dc, vslp, B)
    tsnap = tsnap_flat.reshape(B, MAX_ITER, N)
    conv = conv_lanes.reshape(B, MAX_ITER, L).min(axis=2)
    return _epilogue(tsnap, conv)
